# Initial kernel scaffold; baseline (speedup 1.0000x reference)
import jax
import jax.numpy as jnp
from jax.experimental import pallas as pl


def kernel(x, edge_index, edge_type, W_rel, W_root, b_rgcn, Wq, Wk, Wv, bq, bk, bv, W_skip, b_skip, ln_gamma, ln_beta):
    raise NotImplementedError("write your pallas kernel here")


# speedup vs baseline: 4.0939x; 4.0939x over previous
import functools

import jax
import jax.numpy as jnp
from jax import lax
from jax.experimental import pallas as pl
from jax.experimental.pallas import tpu as pltpu
from jax.experimental.pallas import tpu_sc as plsc

N = 10000
D = 256
R = 4
E = 160000
NC = 2
NS = 16
NW = NC * NS
EPAD = 160256
EW = EPAD // NW
EC = EPAD // NS
BN = 2000

_SC_PARAMS = pltpu.CompilerParams(needs_layout_passes=False)


def _sc_mesh():
    return plsc.VectorSubcoreMesh(core_axis_name="c", subcore_axis_name="s")


def _tca_body(x_ref, w_ref, y_ref):
    y_ref[...] = jnp.dot(x_ref[...], w_ref[0],
                         preferred_element_type=jnp.float32)[None]


def _tc_a(x, w_all):
    return pl.pallas_call(
        _tca_body,
        grid=(R + 1, N // BN),
        in_specs=[
            pl.BlockSpec((BN, D), lambda r, i: (i, 0)),
            pl.BlockSpec((1, D, D), lambda r, i: (r, 0, 0)),
        ],
        out_specs=pl.BlockSpec((1, BN, D), lambda r, i: (r, i, 0)),
        out_shape=jax.ShapeDtypeStruct((R + 1, N, D), jnp.float32),
    )(x, w_all)


def _sc_counts(epk, zrows):
    @functools.partial(
        pl.kernel,
        out_type=jax.ShapeDtypeStruct((EPAD,), jnp.float32),
        mesh=_sc_mesh(),
        compiler_params=_SC_PARAMS,
        scratch_types=[
            pltpu.VMEM((EC,), jnp.int32),
            pltpu.VMEM((320, 128), jnp.float32),
            pltpu.VMEM((EW,), jnp.float32),
            pltpu.VMEM_SHARED((320, 128), jnp.float32),
            pltpu.SemaphoreType.DMA,
        ],
    )
    def k(ep_hbm, z_hbm, w_hbm, ep_v, cnt_v, wbuf, cnt_s, sem_a):
        c = lax.axis_index("c")
        s = lax.axis_index("s")
        wid = c * NS + s
        base = s * EC
        iot = lax.iota(jnp.int32, 16)
        pltpu.sync_copy(ep_hbm.at[pl.ds(base, EC)], ep_v)
        pltpu.sync_copy(z_hbm.at[pl.ds(0, 320)], cnt_v)

        @pl.when(s < 8)
        def _():
            pltpu.sync_copy(z_hbm.at[pl.ds(0, 40)],
                            cnt_s.at[pl.ds(s * 40, 40)])

        plsc.subcore_barrier()
        ones = jnp.ones((16,), jnp.float32)

        def cbody(g, carry):
            ep = ep_v[pl.ds(g * 16, 16)]
            seg = (jnp.right_shift(ep, 16) * R
                   + jnp.bitwise_and(ep, 3))
            eidx = base + g * 16 + iot
            plsc.addupdate_scatter(
                cnt_v,
                [jnp.right_shift(seg, 7), jnp.bitwise_and(seg, 127)],
                ones, mask=eidx < E)
            return carry

        lax.fori_loop(0, EC // 16, cbody, 0)
        for t in range(20):
            pltpu.async_copy(cnt_v.at[pl.ds(t * 16, 16)],
                             cnt_s.at[iot + t * 16], sem_a, add=True).wait()
        plsc.subcore_barrier()
        pltpu.sync_copy(cnt_s, cnt_v)
        wbase = wid * EW
        pltpu.sync_copy(ep_hbm.at[pl.ds(wbase, EW)], ep_v.at[pl.ds(0, EW)])

        def wbody(g, carry):
            ep = ep_v[pl.ds(g * 16, 16)]
            seg = (jnp.right_shift(ep, 16) * R
                   + jnp.bitwise_and(ep, 3))
            cntv = plsc.load_gather(
                cnt_v,
                [jnp.right_shift(seg, 7), jnp.bitwise_and(seg, 127)])
            eidx = wbase + g * 16 + iot
            maskf = jnp.where(eidx < E, 1.0, 0.0)
            wbuf[pl.ds(g * 16, 16)] = maskf / jnp.maximum(cntv, 1.0)
            return carry

        lax.fori_loop(0, EW // 16, wbody, 0)
        pltpu.sync_copy(wbuf, w_hbm.at[pl.ds(wbase, EW)])

    return k(epk, zrows)


def _sc_rgcn(y2, epk, wgt, zrows):
    @functools.partial(
        pl.kernel,
        out_type=jax.ShapeDtypeStruct((NC, N, 128), jnp.float32),
        mesh=_sc_mesh(),
        compiler_params=_SC_PARAMS,
        scratch_types=[
            pltpu.VMEM((EC,), jnp.int32),
            pltpu.VMEM((EC,), jnp.float32),
            pltpu.VMEM((16, 128), jnp.float32),
            pltpu.VMEM_SHARED((N, 128), jnp.float32),
            pltpu.SemaphoreType.DMA,
            pltpu.SemaphoreType.DMA,
        ],
    )
    def k(y_hbm, ep_hbm, w_hbm, z_hbm, out_hbm,
          ep_v, w_v, gbuf, acc_s, sem_g, sem_a):
        c = lax.axis_index("c")
        s = lax.axis_index("s")
        base = s * EC
        iot = lax.iota(jnp.int32, 16)
        pltpu.sync_copy(ep_hbm.at[pl.ds(base, EC)], ep_v)
        pltpu.sync_copy(w_hbm.at[pl.ds(base, EC)], w_v)

        @pl.when(s < 15)
        def _():
            pltpu.sync_copy(z_hbm.at[pl.ds(0, 632)],
                            acc_s.at[pl.ds(s * 632, 632)])

        @pl.when(s == 15)
        def _():
            pltpu.sync_copy(z_hbm.at[pl.ds(0, 520)],
                            acc_s.at[pl.ds(15 * 632, 520)])

        plsc.subcore_barrier()

        def body(g, carry):
            ep = ep_v[pl.ds(g * 16, 16)]
            w = w_v[pl.ds(g * 16, 16)]
            dd = jnp.right_shift(ep, 16)
            e1 = jnp.bitwise_and(ep, 65535)
            row = (jnp.bitwise_and(e1, 3) * (2 * N)
                   + jnp.right_shift(e1, 2) * 2 + c)
            pltpu.async_copy(y_hbm.at[row], gbuf, sem_g).wait()
            for j in range(16):
                wj = jnp.sum(jnp.where(iot == j, w, 0.0))
                for t in range(8):
                    gbuf[j, pl.ds(t * 16, 16)] = (
                        gbuf[j, pl.ds(t * 16, 16)] * wj)
            pltpu.async_copy(gbuf, acc_s.at[dd], sem_a, add=True).wait()
            return carry

        lax.fori_loop(0, EC // 16, body, 0)
        plsc.subcore_barrier()

        @pl.when(s == 0)
        def _():
            pltpu.sync_copy(acc_s, out_hbm.at[c])

    return k(y2, epk, wgt, zrows)


def _sc_logits(qm, km, epk):
    @functools.partial(
        pl.kernel,
        out_type=[jax.ShapeDtypeStruct((EPAD,), jnp.float32),
                  jax.ShapeDtypeStruct((NW * 16,), jnp.float32)],
        mesh=_sc_mesh(),
        compiler_params=_SC_PARAMS,
        scratch_types=[
            pltpu.VMEM((EW,), jnp.int32),
            pltpu.VMEM((16, D), jnp.float32),
            pltpu.VMEM((16, D), jnp.float32),
            pltpu.VMEM((EW,), jnp.float32),
            pltpu.VMEM((16,), jnp.float32),
            pltpu.SemaphoreType.DMA,
            pltpu.SemaphoreType.DMA,
        ],
    )
    def k(q_hbm, k_hbm, ep_hbm, lg_hbm, mx_hbm,
          ep_v, qbuf, kbuf, lbuf, mv, sem_q, sem_k):
        c = lax.axis_index("c")
        s = lax.axis_index("s")
        wid = c * NS + s
        base = wid * EW
        iot = lax.iota(jnp.int32, 16)
        pltpu.sync_copy(ep_hbm.at[pl.ds(base, EW)], ep_v)

        def body(g, m):
            ep = ep_v[pl.ds(g * 16, 16)]
            dd = jnp.right_shift(ep, 16)
            ss = jnp.right_shift(jnp.bitwise_and(ep, 65535), 2)
            cq = pltpu.async_copy(q_hbm.at[dd], qbuf, sem_q)
            ck = pltpu.async_copy(k_hbm.at[ss], kbuf, sem_k)
            cq.wait()
            ck.wait()
            lv = jnp.zeros((16,), jnp.float32)
            for j in range(16):
                acc = jnp.zeros((16,), jnp.float32)
                for t in range(16):
                    acc = acc + (qbuf[j, pl.ds(t * 16, 16)]
                                 * kbuf[j, pl.ds(t * 16, 16)])
                sj = jnp.sum(acc) * 0.0625
                lv = jnp.where(iot == j, sj, lv)
            lbuf[pl.ds(g * 16, 16)] = lv
            return jnp.maximum(m, lv)

        m = lax.fori_loop(0, EW // 16, body,
                          jnp.full((16,), -1e30, jnp.float32))
        mv[...] = m
        pltpu.sync_copy(lbuf, lg_hbm.at[pl.ds(base, EW)])
        pltpu.sync_copy(mv, mx_hbm.at[pl.ds(wid * 16, 16)])

    return k(qm, km, epk)


def _sc_attn(v2, lg, mx, epk, zrows):
    @functools.partial(
        pl.kernel,
        out_type=jax.ShapeDtypeStruct((NC, N, 128), jnp.float32),
        mesh=_sc_mesh(),
        compiler_params=_SC_PARAMS,
        scratch_types=[
            pltpu.VMEM((EC,), jnp.int32),
            pltpu.VMEM((EC,), jnp.float32),
            pltpu.VMEM((80, 128), jnp.float32),
            pltpu.VMEM((NW * 16,), jnp.float32),
            pltpu.VMEM((16, 128), jnp.float32),
            pltpu.VMEM_SHARED((80, 128), jnp.float32),
            pltpu.VMEM_SHARED((N, 128), jnp.float32),
            pltpu.SemaphoreType.DMA,
            pltpu.SemaphoreType.DMA,
        ],
    )
    def k(v_hbm, lg_hbm, mx_hbm, ep_hbm, z_hbm, out_hbm,
          ep_v, lg_v, den_v, mxv, gbuf, den_s, acc_s, sem_g, sem_a):
        c = lax.axis_index("c")
        s = lax.axis_index("s")
        base = s * EC
        iot = lax.iota(jnp.int32, 16)
        pltpu.sync_copy(ep_hbm.at[pl.ds(base, EC)], ep_v)
        pltpu.sync_copy(lg_hbm.at[pl.ds(base, EC)], lg_v)
        pltpu.sync_copy(mx_hbm, mxv)
        pltpu.sync_copy(z_hbm.at[pl.ds(0, 80)], den_v)

        @pl.when(s < 10)
        def _():
            pltpu.sync_copy(z_hbm.at[pl.ds(0, 8)],
                            den_s.at[pl.ds(s * 8, 8)])

        @pl.when(s < 15)
        def _():
            pltpu.sync_copy(z_hbm.at[pl.ds(0, 632)],
                            acc_s.at[pl.ds(s * 632, 632)])

        @pl.when(s == 15)
        def _():
            pltpu.sync_copy(z_hbm.at[pl.ds(0, 520)],
                            acc_s.at[pl.ds(15 * 632, 520)])

        m = jnp.full((16,), -1e30, jnp.float32)
        for i in range(NW):
            m = jnp.maximum(m, mxv[pl.ds(i * 16, 16)])
        gmax = jnp.max(m)
        plsc.subcore_barrier()

        def dbody(g, carry):
            ep = ep_v[pl.ds(g * 16, 16)]
            dd = jnp.right_shift(ep, 16)
            l = lg_v[pl.ds(g * 16, 16)]
            e = jnp.exp(l - gmax)
            eidx = base + g * 16 + iot
            plsc.addupdate_scatter(
                den_v,
                [jnp.right_shift(dd, 7), jnp.bitwise_and(dd, 127)],
                e, mask=eidx < E)
            return carry

        lax.fori_loop(0, EC // 16, dbody, 0)
        for t in range(5):
            pltpu.async_copy(den_v.at[pl.ds(t * 16, 16)],
                             den_s.at[iot + t * 16], sem_a, add=True).wait()
        plsc.subcore_barrier()
        pltpu.sync_copy(den_s, den_v)

        def body(g, carry):
            ep = ep_v[pl.ds(g * 16, 16)]
            dd = jnp.right_shift(ep, 16)
            ss = jnp.right_shift(jnp.bitwise_and(ep, 65535), 2)
            l = lg_v[pl.ds(g * 16, 16)]
            e = jnp.exp(l - gmax)
            dn = plsc.load_gather(
                den_v,
                [jnp.right_shift(dd, 7), jnp.bitwise_and(dd, 127)])
            eidx = base + g * 16 + iot
            maskf = jnp.where(eidx < E, 1.0, 0.0)
            w = e * maskf / jnp.maximum(dn, 1e-16)
            row = ss * 2 + c
            pltpu.async_copy(v_hbm.at[row], gbuf, sem_g).wait()
            for j in range(16):
                wj = jnp.sum(jnp.where(iot == j, w, 0.0))
                for t in range(8):
                    gbuf[j, pl.ds(t * 16, 16)] = (
                        gbuf[j, pl.ds(t * 16, 16)] * wj)
            pltpu.async_copy(gbuf, acc_s.at[dd], sem_a, add=True).wait()
            return carry

        lax.fori_loop(0, EC // 16, body, 0)
        plsc.subcore_barrier()

        @pl.when(s == 0)
        def _():
            pltpu.sync_copy(acc_s, out_hbm.at[c])

    return k(v2, lg, mx, epk, zrows)


def _tcb_body(yr_ref, br_ref, agg_ref, wq_ref, bq_ref, wk_ref, bk_ref,
              wv_ref, bv_ref, ws_ref, bs_ref,
              q_ref, k_ref, v_ref, hs_ref):
    a = agg_ref[...]
    h = (yr_ref[...][0] + br_ref[0]
         + jnp.concatenate([a[0], a[1]], axis=-1))
    q_ref[...] = jnp.dot(h, wq_ref[...],
                         preferred_element_type=jnp.float32) + bq_ref[0]
    k_ref[...] = jnp.dot(h, wk_ref[...],
                         preferred_element_type=jnp.float32) + bk_ref[0]
    v_ref[...] = jnp.dot(h, wv_ref[...],
                         preferred_element_type=jnp.float32) + bv_ref[0]
    hs_ref[...] = jnp.dot(h, ws_ref[...],
                          preferred_element_type=jnp.float32) + bs_ref[0]


def _tc_b(y, br, agg, wq, bq, wk, bk, wv, bv, ws, bs):
    full = lambda i: (0, 0)
    return pl.pallas_call(
        _tcb_body,
        grid=(N // BN,),
        in_specs=[
            pl.BlockSpec((1, BN, D), lambda i: (R, i, 0)),
            pl.BlockSpec((1, D), full),
            pl.BlockSpec((NC, BN, 128), lambda i: (0, i, 0)),
            pl.BlockSpec((D, D), full), pl.BlockSpec((1, D), full),
            pl.BlockSpec((D, D), full), pl.BlockSpec((1, D), full),
            pl.BlockSpec((D, D), full), pl.BlockSpec((1, D), full),
            pl.BlockSpec((D, D), full), pl.BlockSpec((1, D), full),
        ],
        out_specs=[pl.BlockSpec((BN, D), lambda i: (i, 0))] * 4,
        out_shape=[jax.ShapeDtypeStruct((N, D), jnp.float32)] * 4,
    )(y, br, agg, wq, bq, wk, bk, wv, bv, ws, bs)


def _tcc_body(x_ref, oa_ref, hs_ref, g_ref, b_ref, o_ref):
    a = oa_ref[...]
    y = x_ref[...] + hs_ref[...] + jnp.concatenate([a[0], a[1]], axis=-1)
    mu = jnp.mean(y, axis=-1, keepdims=True)
    yc = y - mu
    var = jnp.mean(yc * yc, axis=-1, keepdims=True)
    o_ref[...] = g_ref[0] * (yc * lax.rsqrt(var + 1e-5)) + b_ref[0]


def _tc_c(x, oa, hs, g, b):
    full = lambda i: (0, 0)
    return pl.pallas_call(
        _tcc_body,
        grid=(N // BN,),
        in_specs=[
            pl.BlockSpec((BN, D), lambda i: (i, 0)),
            pl.BlockSpec((NC, BN, 128), lambda i: (0, i, 0)),
            pl.BlockSpec((BN, D), lambda i: (i, 0)),
            pl.BlockSpec((1, D), full),
            pl.BlockSpec((1, D), full),
        ],
        out_specs=pl.BlockSpec((BN, D), lambda i: (i, 0)),
        out_shape=jax.ShapeDtypeStruct((N, D), jnp.float32),
    )(x, oa, hs, g, b)


def kernel(x, edge_index, edge_type, W_rel, W_root, b_rgcn, Wq, Wk, Wv,
           bq, bk, bv, W_skip, b_skip, ln_gamma, ln_beta):
    src = edge_index[0].astype(jnp.int32)
    dst = edge_index[1].astype(jnp.int32)
    rel = edge_type.astype(jnp.int32)
    packed = dst * 65536 + src * 4 + rel
    epk = jnp.concatenate([packed, jnp.zeros((EPAD - E,), jnp.int32)])
    zrows = jnp.zeros((N, 128), jnp.float32)
    w_all = jnp.concatenate([W_rel, W_root[None]], axis=0)

    y = _tc_a(x, w_all)
    y2 = y.reshape((R + 1) * N * 2, 128)
    wgt = _sc_counts(epk, zrows)
    agg = _sc_rgcn(y2, epk, wgt, zrows)
    q, kk, v, hs = _tc_b(y, b_rgcn.reshape(1, D), agg,
                         Wq, bq.reshape(1, D), Wk, bk.reshape(1, D),
                         Wv, bv.reshape(1, D), W_skip, b_skip.reshape(1, D))
    lg, mx = _sc_logits(q, kk, epk)
    oa = _sc_attn(v.reshape(N * 2, 128), lg, mx, epk, zrows)
    return _tc_c(x, oa, hs, ln_gamma.reshape(1, D), ln_beta.reshape(1, D))


# speedup vs baseline: 6.3513x; 1.5514x over previous
import functools

import jax
import jax.numpy as jnp
from jax import lax
from jax.experimental import pallas as pl
from jax.experimental.pallas import tpu as pltpu
from jax.experimental.pallas import tpu_sc as plsc

N = 10000
D = 256
R = 4
E = 160000
NC = 2
NS = 16
NW = NC * NS
EPAD = 160768
EW = EPAD // NW
EC = EPAD // NS
BN = 2000

_SC_PARAMS = pltpu.CompilerParams(needs_layout_passes=False)


def _sc_mesh():
    return plsc.VectorSubcoreMesh(core_axis_name="c", subcore_axis_name="s")


def _tca_body(x_ref, w_ref, y_ref):
    y_ref[...] = jnp.dot(x_ref[...], w_ref[0],
                         preferred_element_type=jnp.float32)[None]


def _tc_a(x, w_all):
    return pl.pallas_call(
        _tca_body,
        grid=(R + 1, N // BN),
        in_specs=[
            pl.BlockSpec((BN, D), lambda r, i: (i, 0)),
            pl.BlockSpec((1, D, D), lambda r, i: (r, 0, 0)),
        ],
        out_specs=pl.BlockSpec((1, BN, D), lambda r, i: (r, i, 0)),
        out_shape=jax.ShapeDtypeStruct((R + 1, N, D), jnp.float32),
    )(x, w_all)


def _sc_counts(epk, zrows):
    @functools.partial(
        pl.kernel,
        out_type=jax.ShapeDtypeStruct((EPAD,), jnp.float32),
        mesh=_sc_mesh(),
        compiler_params=_SC_PARAMS,
        scratch_types=[
            pltpu.VMEM((EC,), jnp.int32),
            pltpu.VMEM((320, 128), jnp.float32),
            pltpu.VMEM((EW,), jnp.float32),
            pltpu.VMEM_SHARED((320, 128), jnp.float32),
            pltpu.SemaphoreType.DMA,
        ],
    )
    def k(ep_hbm, z_hbm, w_hbm, ep_v, cnt_v, wbuf, cnt_s, sem_a):
        c = lax.axis_index("c")
        s = lax.axis_index("s")
        wid = c * NS + s
        base = s * EC
        iot = lax.iota(jnp.int32, 16)
        pltpu.sync_copy(ep_hbm.at[pl.ds(base, EC)], ep_v)
        pltpu.sync_copy(z_hbm.at[pl.ds(0, 320)], cnt_v)

        @pl.when(s < 8)
        def _():
            pltpu.sync_copy(z_hbm.at[pl.ds(0, 40)],
                            cnt_s.at[pl.ds(s * 40, 40)])

        plsc.subcore_barrier()
        ones = jnp.ones((16,), jnp.float32)

        def cbody(g, carry):
            ep = ep_v[pl.ds(g * 16, 16)]
            seg = (jnp.right_shift(ep, 16) * R
                   + jnp.bitwise_and(ep, 3))
            eidx = base + g * 16 + iot
            plsc.addupdate_scatter(
                cnt_v,
                [jnp.right_shift(seg, 7), jnp.bitwise_and(seg, 127)],
                ones, mask=eidx < E)
            return carry

        lax.fori_loop(0, EC // 16, cbody, 0)
        for t in range(20):
            pltpu.async_copy(cnt_v.at[pl.ds(t * 16, 16)],
                             cnt_s.at[iot + t * 16], sem_a, add=True).wait()
        plsc.subcore_barrier()
        pltpu.sync_copy(cnt_s, cnt_v)
        wbase = wid * EW
        pltpu.sync_copy(ep_hbm.at[pl.ds(wbase, EW)], ep_v.at[pl.ds(0, EW)])

        def wbody(g, carry):
            ep = ep_v[pl.ds(g * 16, 16)]
            seg = (jnp.right_shift(ep, 16) * R
                   + jnp.bitwise_and(ep, 3))
            cntv = plsc.load_gather(
                cnt_v,
                [jnp.right_shift(seg, 7), jnp.bitwise_and(seg, 127)])
            eidx = wbase + g * 16 + iot
            maskf = jnp.where(eidx < E, 1.0, 0.0)
            wbuf[pl.ds(g * 16, 16)] = maskf / jnp.maximum(cntv, 1.0)
            return carry

        lax.fori_loop(0, EW // 16, wbody, 0)
        pltpu.sync_copy(wbuf, w_hbm.at[pl.ds(wbase, EW)])

    return k(epk, zrows)


def _sc_rgcn(y2, epk, wgt, zrows):
    @functools.partial(
        pl.kernel,
        out_type=jax.ShapeDtypeStruct((NC, N, 128), jnp.float32),
        mesh=_sc_mesh(),
        compiler_params=_SC_PARAMS,
        scratch_types=[
            pltpu.VMEM((EC,), jnp.int32),
            pltpu.VMEM((EC,), jnp.float32),
            pltpu.VMEM((16, 128), jnp.float32),
            pltpu.VMEM((16, 128), jnp.float32),
            pltpu.VMEM((16,), jnp.int32),
            pltpu.VMEM((16,), jnp.int32),
            pltpu.VMEM((16,), jnp.int32),
            pltpu.VMEM((16,), jnp.int32),
            pltpu.VMEM_SHARED((N, 128), jnp.float32),
            pltpu.SemaphoreType.DMA,
            pltpu.SemaphoreType.DMA,
            pltpu.SemaphoreType.DMA,
            pltpu.SemaphoreType.DMA,
        ],
    )
    def k(y_hbm, ep_hbm, w_hbm, z_hbm, out_hbm,
          ep_v, w_v, gbuf0, gbuf1, gi0, gi1, si0, si1, acc_s,
          gs0, gs1, ss0, ss1):
        c = lax.axis_index("c")
        s = lax.axis_index("s")
        base = s * EC
        iot = lax.iota(jnp.int32, 16)
        gbufs, gsem, ssem = (gbuf0, gbuf1), (gs0, gs1), (ss0, ss1)
        gidx, sidx = (gi0, gi1), (si0, si1)
        pltpu.sync_copy(ep_hbm.at[pl.ds(base, EC)], ep_v)
        pltpu.sync_copy(w_hbm.at[pl.ds(base, EC)], w_v)

        @pl.when(s < 15)
        def _():
            pltpu.sync_copy(z_hbm.at[pl.ds(0, 632)],
                            acc_s.at[pl.ds(s * 632, 632)])

        @pl.when(s == 15)
        def _():
            pltpu.sync_copy(z_hbm.at[pl.ds(0, 520)],
                            acc_s.at[pl.ds(15 * 632, 520)])

        plsc.subcore_barrier()
        ngrp = EC // 16

        def issue_gather(g, b):
            ep = ep_v[pl.ds(g * 16, 16)]
            e1 = jnp.bitwise_and(ep, 65535)
            gidx[b][...] = (jnp.bitwise_and(e1, 3) * (2 * N)
                            + jnp.right_shift(e1, 2) * 2 + c)
            pltpu.async_copy(y_hbm.at[gidx[b]], gbufs[b], gsem[b])

        issue_gather(0, 0)

        def pair(p, carry):
            for b in range(2):
                g = p * 2 + b
                nb = 1 - b

                @pl.when(g + 1 < ngrp)
                def _():
                    @pl.when(g >= 1)
                    def _():
                        pltpu.make_async_copy(
                            gbufs[nb], acc_s.at[sidx[nb]], ssem[nb]).wait()
                    issue_gather(g + 1, nb)

                pltpu.make_async_copy(y_hbm.at[gidx[b]],
                                      gbufs[b], gsem[b]).wait()
                w = w_v[pl.ds(g * 16, 16)]
                sidx[b][...] = jnp.right_shift(ep_v[pl.ds(g * 16, 16)], 16)
                for j in range(16):
                    wj = jnp.sum(jnp.where(iot == j, w, 0.0))
                    for t in range(8):
                        gbufs[b][j, pl.ds(t * 16, 16)] = (
                            gbufs[b][j, pl.ds(t * 16, 16)] * wj)
                pltpu.async_copy(gbufs[b], acc_s.at[sidx[b]], ssem[b],
                                 add=True)
            return carry

        lax.fori_loop(0, ngrp // 2, pair, 0)
        for b in range(2):
            pltpu.make_async_copy(gbufs[b], acc_s.at[sidx[b]],
                                  ssem[b]).wait()
        plsc.subcore_barrier()

        @pl.when(s == 0)
        def _():
            pltpu.sync_copy(acc_s, out_hbm.at[c])

    return k(y2, epk, wgt, zrows)


def _sc_logits(qm, km, epk):
    @functools.partial(
        pl.kernel,
        out_type=[jax.ShapeDtypeStruct((EPAD,), jnp.float32),
                  jax.ShapeDtypeStruct((NW * 16,), jnp.float32)],
        mesh=_sc_mesh(),
        compiler_params=_SC_PARAMS,
        scratch_types=[
            pltpu.VMEM((EW,), jnp.int32),
            pltpu.VMEM((16, D), jnp.float32),
            pltpu.VMEM((16, D), jnp.float32),
            pltpu.VMEM((16, D), jnp.float32),
            pltpu.VMEM((16, D), jnp.float32),
            pltpu.VMEM((EW,), jnp.float32),
            pltpu.VMEM((16,), jnp.float32),
            pltpu.VMEM((16,), jnp.int32),
            pltpu.VMEM((16,), jnp.int32),
            pltpu.VMEM((16,), jnp.int32),
            pltpu.VMEM((16,), jnp.int32),
            pltpu.SemaphoreType.DMA,
            pltpu.SemaphoreType.DMA,
            pltpu.SemaphoreType.DMA,
            pltpu.SemaphoreType.DMA,
        ],
    )
    def k(q_hbm, k_hbm, ep_hbm, lg_hbm, mx_hbm,
          ep_v, qbuf0, qbuf1, kbuf0, kbuf1, lbuf, mv,
          qi0, qi1, ki0, ki1, qs0, qs1, ks0, ks1):
        c = lax.axis_index("c")
        s = lax.axis_index("s")
        wid = c * NS + s
        base = wid * EW
        iot = lax.iota(jnp.int32, 16)
        qbufs, kbufs = (qbuf0, qbuf1), (kbuf0, kbuf1)
        qsem, ksem = (qs0, qs1), (ks0, ks1)
        qidx, kidx = (qi0, qi1), (ki0, ki1)
        pltpu.sync_copy(ep_hbm.at[pl.ds(base, EW)], ep_v)
        ngrp = EW // 16

        def issue_gather(g, b):
            ep = ep_v[pl.ds(g * 16, 16)]
            qidx[b][...] = jnp.right_shift(ep, 16)
            kidx[b][...] = jnp.right_shift(jnp.bitwise_and(ep, 65535), 2)
            pltpu.async_copy(q_hbm.at[qidx[b]], qbufs[b], qsem[b])
            pltpu.async_copy(k_hbm.at[kidx[b]], kbufs[b], ksem[b])

        issue_gather(0, 0)

        def pair(p, m):
            for b in range(2):
                g = p * 2 + b
                nb = 1 - b

                @pl.when(g + 1 < ngrp)
                def _():
                    issue_gather(g + 1, nb)

                pltpu.make_async_copy(q_hbm.at[qidx[b]],
                                      qbufs[b], qsem[b]).wait()
                pltpu.make_async_copy(k_hbm.at[kidx[b]],
                                      kbufs[b], ksem[b]).wait()
                lv = jnp.zeros((16,), jnp.float32)
                for j in range(16):
                    acc = jnp.zeros((16,), jnp.float32)
                    for t in range(16):
                        acc = acc + (qbufs[b][j, pl.ds(t * 16, 16)]
                                     * kbufs[b][j, pl.ds(t * 16, 16)])
                    sj = jnp.sum(acc) * 0.0625
                    lv = jnp.where(iot == j, sj, lv)
                lbuf[pl.ds(g * 16, 16)] = lv
                m = jnp.maximum(m, lv)
            return m

        m = lax.fori_loop(0, ngrp // 2, pair,
                          jnp.full((16,), -1e30, jnp.float32))
        mv[...] = m
        pltpu.sync_copy(lbuf, lg_hbm.at[pl.ds(base, EW)])
        pltpu.sync_copy(mv, mx_hbm.at[pl.ds(wid * 16, 16)])

    return k(qm, km, epk)


def _sc_attn(v2, lg, mx, epk, zrows):
    @functools.partial(
        pl.kernel,
        out_type=jax.ShapeDtypeStruct((NC, N, 128), jnp.float32),
        mesh=_sc_mesh(),
        compiler_params=_SC_PARAMS,
        scratch_types=[
            pltpu.VMEM((EC,), jnp.int32),
            pltpu.VMEM((EC,), jnp.float32),
            pltpu.VMEM((80, 128), jnp.float32),
            pltpu.VMEM((NW * 16,), jnp.float32),
            pltpu.VMEM((16, 128), jnp.float32),
            pltpu.VMEM((16, 128), jnp.float32),
            pltpu.VMEM((16,), jnp.int32),
            pltpu.VMEM((16,), jnp.int32),
            pltpu.VMEM((16,), jnp.int32),
            pltpu.VMEM((16,), jnp.int32),
            pltpu.VMEM_SHARED((80, 128), jnp.float32),
            pltpu.VMEM_SHARED((N, 128), jnp.float32),
            pltpu.SemaphoreType.DMA,
            pltpu.SemaphoreType.DMA,
            pltpu.SemaphoreType.DMA,
            pltpu.SemaphoreType.DMA,
        ],
    )
    def k(v_hbm, lg_hbm, mx_hbm, ep_hbm, z_hbm, out_hbm,
          ep_v, lg_v, den_v, mxv, gbuf0, gbuf1, gi0, gi1, si0, si1,
          den_s, acc_s, gs0, gs1, ss0, ss1):
        gbufs, gsem, ssem = (gbuf0, gbuf1), (gs0, gs1), (ss0, ss1)
        gidx, sidx = (gi0, gi1), (si0, si1)
        c = lax.axis_index("c")
        s = lax.axis_index("s")
        base = s * EC
        iot = lax.iota(jnp.int32, 16)
        pltpu.sync_copy(ep_hbm.at[pl.ds(base, EC)], ep_v)
        pltpu.sync_copy(lg_hbm.at[pl.ds(base, EC)], lg_v)
        pltpu.sync_copy(mx_hbm, mxv)
        pltpu.sync_copy(z_hbm.at[pl.ds(0, 80)], den_v)

        @pl.when(s < 10)
        def _():
            pltpu.sync_copy(z_hbm.at[pl.ds(0, 8)],
                            den_s.at[pl.ds(s * 8, 8)])

        @pl.when(s < 15)
        def _():
            pltpu.sync_copy(z_hbm.at[pl.ds(0, 632)],
                            acc_s.at[pl.ds(s * 632, 632)])

        @pl.when(s == 15)
        def _():
            pltpu.sync_copy(z_hbm.at[pl.ds(0, 520)],
                            acc_s.at[pl.ds(15 * 632, 520)])

        m = jnp.full((16,), -1e30, jnp.float32)
        for i in range(NW):
            m = jnp.maximum(m, mxv[pl.ds(i * 16, 16)])
        gmax = jnp.max(m)
        plsc.subcore_barrier()

        def dbody(g, carry):
            ep = ep_v[pl.ds(g * 16, 16)]
            dd = jnp.right_shift(ep, 16)
            l = lg_v[pl.ds(g * 16, 16)]
            e = jnp.exp(l - gmax)
            eidx = base + g * 16 + iot
            plsc.addupdate_scatter(
                den_v,
                [jnp.right_shift(dd, 7), jnp.bitwise_and(dd, 127)],
                e, mask=eidx < E)
            return carry

        lax.fori_loop(0, EC // 16, dbody, 0)
        for t in range(5):
            pltpu.async_copy(den_v.at[pl.ds(t * 16, 16)],
                             den_s.at[iot + t * 16], gs0, add=True).wait()
        plsc.subcore_barrier()
        pltpu.sync_copy(den_s, den_v)
        ngrp = EC // 16

        def issue_gather(g, b):
            ep = ep_v[pl.ds(g * 16, 16)]
            gidx[b][...] = (jnp.right_shift(jnp.bitwise_and(ep, 65535), 2)
                            * 2 + c)
            pltpu.async_copy(v_hbm.at[gidx[b]], gbufs[b], gsem[b])

        issue_gather(0, 0)

        def pair(p, carry):
            for b in range(2):
                g = p * 2 + b
                nb = 1 - b

                @pl.when(g + 1 < ngrp)
                def _():
                    @pl.when(g >= 1)
                    def _():
                        pltpu.make_async_copy(
                            gbufs[nb], acc_s.at[sidx[nb]], ssem[nb]).wait()
                    issue_gather(g + 1, nb)

                pltpu.make_async_copy(v_hbm.at[gidx[b]],
                                      gbufs[b], gsem[b]).wait()
                ep = ep_v[pl.ds(g * 16, 16)]
                dd = jnp.right_shift(ep, 16)
                l = lg_v[pl.ds(g * 16, 16)]
                e = jnp.exp(l - gmax)
                dn = plsc.load_gather(
                    den_v,
                    [jnp.right_shift(dd, 7), jnp.bitwise_and(dd, 127)])
                eidx = base + g * 16 + iot
                maskf = jnp.where(eidx < E, 1.0, 0.0)
                w = e * maskf / jnp.maximum(dn, 1e-16)
                sidx[b][...] = dd
                for j in range(16):
                    wj = jnp.sum(jnp.where(iot == j, w, 0.0))
                    for t in range(8):
                        gbufs[b][j, pl.ds(t * 16, 16)] = (
                            gbufs[b][j, pl.ds(t * 16, 16)] * wj)
                pltpu.async_copy(gbufs[b], acc_s.at[sidx[b]], ssem[b],
                                 add=True)
            return carry

        lax.fori_loop(0, ngrp // 2, pair, 0)
        for b in range(2):
            pltpu.make_async_copy(gbufs[b], acc_s.at[sidx[b]],
                                  ssem[b]).wait()
        plsc.subcore_barrier()

        @pl.when(s == 0)
        def _():
            pltpu.sync_copy(acc_s, out_hbm.at[c])

    return k(v2, lg, mx, epk, zrows)


def _tcb_body(yr_ref, br_ref, agg_ref, wq_ref, bq_ref, wk_ref, bk_ref,
              wv_ref, bv_ref, ws_ref, bs_ref,
              q_ref, k_ref, v_ref, hs_ref):
    a = agg_ref[...]
    h = (yr_ref[...][0] + br_ref[0]
         + jnp.concatenate([a[0], a[1]], axis=-1))
    q_ref[...] = jnp.dot(h, wq_ref[...],
                         preferred_element_type=jnp.float32) + bq_ref[0]
    k_ref[...] = jnp.dot(h, wk_ref[...],
                         preferred_element_type=jnp.float32) + bk_ref[0]
    v_ref[...] = jnp.dot(h, wv_ref[...],
                         preferred_element_type=jnp.float32) + bv_ref[0]
    hs_ref[...] = jnp.dot(h, ws_ref[...],
                          preferred_element_type=jnp.float32) + bs_ref[0]


def _tc_b(y, br, agg, wq, bq, wk, bk, wv, bv, ws, bs):
    full = lambda i: (0, 0)
    return pl.pallas_call(
        _tcb_body,
        grid=(N // BN,),
        in_specs=[
            pl.BlockSpec((1, BN, D), lambda i: (R, i, 0)),
            pl.BlockSpec((1, D), full),
            pl.BlockSpec((NC, BN, 128), lambda i: (0, i, 0)),
            pl.BlockSpec((D, D), full), pl.BlockSpec((1, D), full),
            pl.BlockSpec((D, D), full), pl.BlockSpec((1, D), full),
            pl.BlockSpec((D, D), full), pl.BlockSpec((1, D), full),
            pl.BlockSpec((D, D), full), pl.BlockSpec((1, D), full),
        ],
        out_specs=[pl.BlockSpec((BN, D), lambda i: (i, 0))] * 4,
        out_shape=[jax.ShapeDtypeStruct((N, D), jnp.float32)] * 4,
    )(y, br, agg, wq, bq, wk, bk, wv, bv, ws, bs)


def _tcc_body(x_ref, oa_ref, hs_ref, g_ref, b_ref, o_ref):
    a = oa_ref[...]
    y = x_ref[...] + hs_ref[...] + jnp.concatenate([a[0], a[1]], axis=-1)
    mu = jnp.mean(y, axis=-1, keepdims=True)
    yc = y - mu
    var = jnp.mean(yc * yc, axis=-1, keepdims=True)
    o_ref[...] = g_ref[0] * (yc * lax.rsqrt(var + 1e-5)) + b_ref[0]


def _tc_c(x, oa, hs, g, b):
    full = lambda i: (0, 0)
    return pl.pallas_call(
        _tcc_body,
        grid=(N // BN,),
        in_specs=[
            pl.BlockSpec((BN, D), lambda i: (i, 0)),
            pl.BlockSpec((NC, BN, 128), lambda i: (0, i, 0)),
            pl.BlockSpec((BN, D), lambda i: (i, 0)),
            pl.BlockSpec((1, D), full),
            pl.BlockSpec((1, D), full),
        ],
        out_specs=pl.BlockSpec((BN, D), lambda i: (i, 0)),
        out_shape=jax.ShapeDtypeStruct((N, D), jnp.float32),
    )(x, oa, hs, g, b)


def kernel(x, edge_index, edge_type, W_rel, W_root, b_rgcn, Wq, Wk, Wv,
           bq, bk, bv, W_skip, b_skip, ln_gamma, ln_beta):
    src = edge_index[0].astype(jnp.int32)
    dst = edge_index[1].astype(jnp.int32)
    rel = edge_type.astype(jnp.int32)
    packed = dst * 65536 + src * 4 + rel
    epk = jnp.concatenate([packed, jnp.zeros((EPAD - E,), jnp.int32)])
    zrows = jnp.zeros((N, 128), jnp.float32)
    w_all = jnp.concatenate([W_rel, W_root[None]], axis=0)

    y = _tc_a(x, w_all)
    y2 = y.reshape((R + 1) * N * 2, 128)
    wgt = _sc_counts(epk, zrows)
    agg = _sc_rgcn(y2, epk, wgt, zrows)
    q, kk, v, hs = _tc_b(y, b_rgcn.reshape(1, D), agg,
                         Wq, bq.reshape(1, D), Wk, bk.reshape(1, D),
                         Wv, bv.reshape(1, D), W_skip, b_skip.reshape(1, D))
    lg, mx = _sc_logits(q, kk, epk)
    oa = _sc_attn(v.reshape(N * 2, 128), lg, mx, epk, zrows)
    return _tc_c(x, oa, hs, ln_gamma.reshape(1, D), ln_beta.reshape(1, D))
